# baseline (speedup 1.0000x reference)
import functools

import jax
import jax.numpy as jnp
from jax import lax
from jax.experimental import pallas as pl
from jax.experimental.pallas import tpu as pltpu
from jax.experimental.pallas import tpu_sc as plsc

B = 16384
F = 26
EMBD = 32
N = B * F

NC = 2
NS = 16
NW = NC * NS
CHUNK = 128
FT = 4
NBB = B // CHUNK
RAW = FT * NBB * 8
A_ROWS = 3 * NBB * 8
A_PER_W = A_ROWS // NW
B_PER_W = (RAW - A_ROWS) // NW
SUP = 4
N_SUP = 26
LANES = 16


def _raw_view(x):
    return (
        jnp.pad(x.T, ((0, 6), (0, 0)))
        .reshape(FT, 8, NBB, CHUNK)
        .transpose(0, 2, 1, 3)
        .reshape(RAW, CHUNK)
    )


def kernel(xi, xv, ca_emb_weight):
    xi_n = _raw_view(xi.astype(jnp.int32))
    xv_n = _raw_view(xv)

    mesh = plsc.VectorSubcoreMesh(core_axis_name="c", subcore_axis_name="s")

    @functools.partial(
        pl.kernel,
        out_type=jax.ShapeDtypeStruct((F, EMBD // 8, NBB, 8, CHUNK),
                                      jnp.float32),
        mesh=mesh,
        scratch_types=[
            pltpu.VMEM((A_PER_W + B_PER_W, CHUNK), jnp.int32),
            pltpu.VMEM((A_PER_W + B_PER_W, CHUNK), jnp.float32),
            pltpu.VMEM((SUP, CHUNK, EMBD), jnp.float32),
            pltpu.VMEM((SUP, CHUNK, EMBD), jnp.float32),
            pltpu.VMEM((EMBD // 8, SUP, 8, CHUNK), jnp.float32),
            pltpu.VMEM((EMBD // 8, SUP, 8, CHUNK), jnp.float32),
            pltpu.SemaphoreType.DMA((2,)),
            pltpu.SemaphoreType.DMA((2,)),
        ],
        compiler_params=pltpu.CompilerParams(
            use_tc_tiling_on_sc=False, needs_layout_passes=False
        ),
    )
    def k(table_hbm, idx_hbm, xv_hbm, out_hbm,
          idx_v, xv_v, gbuf0, gbuf1, obuf0, obuf1, gsem, wsem):
        gbuf = (gbuf0, gbuf1)
        obuf = (obuf0, obuf1)
        wid = lax.axis_index("s") * NC + lax.axis_index("c")

        pltpu.sync_copy(idx_hbm.at[pl.ds(wid * A_PER_W, A_PER_W)],
                        idx_v.at[pl.ds(0, A_PER_W)])
        pltpu.sync_copy(idx_hbm.at[pl.ds(A_ROWS + wid * B_PER_W, B_PER_W)],
                        idx_v.at[pl.ds(A_PER_W, B_PER_W)])
        pltpu.sync_copy(xv_hbm.at[pl.ds(wid * A_PER_W, A_PER_W)],
                        xv_v.at[pl.ds(0, A_PER_W)])
        pltpu.sync_copy(xv_hbm.at[pl.ds(A_ROWS + wid * B_PER_W, B_PER_W)],
                        xv_v.at[pl.ds(A_PER_W, B_PER_W)])

        iota16 = lax.iota(jnp.int32, LANES)
        jsplat = [jnp.full((LANES,), j, jnp.int32) for j in range(SUP)]
        esplat = [jnp.full((LANES,), e, jnp.int32) for e in range(EMBD)]

        def sup_info(i):
            is_b = i >= 24
            fs = jnp.where(is_b, i - 24, lax.bitwise_and(i, 7))
            g = lax.shift_right_logical(i, 3)
            lb = jnp.where(is_b, A_PER_W, g * 32) + fs
            rr = jnp.where(is_b, A_ROWS + wid * B_PER_W,
                           wid * A_PER_W + g * 32) + fs
            f = lax.shift_right_logical(rr, 10) * 8 + fs
            bb0 = lax.shift_right_logical(lax.bitwise_and(rr, 1023), 3)
            return lb, f, bb0

        def start_gathers(i, b):
            lb, _, _ = sup_info(i)
            for j in range(SUP):
                pltpu.async_copy(
                    table_hbm.at[idx_v.at[lb + 8 * j]],
                    gbuf[b].at[j],
                    gsem.at[b],
                )

        def wait_gathers(i, b):
            lb, _, _ = sup_info(i)
            for j in range(SUP):
                pltpu.make_async_copy(
                    table_hbm.at[idx_v.at[lb + 8 * j]],
                    gbuf[b].at[j],
                    gsem.at[b],
                ).wait()

        def out_slice(i, tr):
            _, f, bb0 = sup_info(i)
            return out_hbm.at[f, tr, pl.ds(bb0, SUP)]

        def start_writebacks(i, b):
            for tr in range(EMBD // 8):
                pltpu.async_copy(obuf[b].at[tr], out_slice(i, tr), wsem.at[b])

        def wait_writebacks(i, b):
            for tr in range(EMBD // 8):
                pltpu.make_async_copy(
                    obuf[b].at[tr], out_slice(i, tr), wsem.at[b]
                ).wait()

        def compute(i, b):
            g_ref, o_ref = gbuf[b], obuf[b]
            lb, _, _ = sup_info(i)

            @plsc.parallel_loop(0, CHUNK // LANES, unroll=2)
            def _(bl0):
                lane0 = bl0 * LANES
                row_idx = lane0 + iota16
                for j in range(SUP):
                    xvv = xv_v[lb + 8 * j, pl.ds(lane0, LANES)]
                    for e in range(EMBD):
                        g = plsc.load_gather(
                            g_ref, [jsplat[j], row_idx, esplat[e]]
                        )
                        o_ref.at[e // 8, j, e % 8, pl.ds(lane0, LANES)][...] = (
                            g * xvv
                        )

        start_gathers(0, 0)

        @pl.loop(0, N_SUP, step=2)
        def _(i0):
            for b in range(2):
                i = i0 + b
                wait_gathers(i, b)
                @pl.when(i + 1 < N_SUP)
                def _():
                    start_gathers(i + 1, 1 - b)
                @pl.when(i >= 2)
                def _():
                    wait_writebacks(i - 2, b)
                compute(i, b)
                start_writebacks(i, b)

        for b in range(2):
            wait_writebacks(N_SUP - 2 + b, b)

    out5d = k(ca_emb_weight, xi_n, xv_n)
    return jnp.transpose(out5d, (2, 4, 0, 1, 3)).reshape(B, F, EMBD)


# speedup vs baseline: 1.0631x; 1.0631x over previous
import functools

import jax
import jax.numpy as jnp
from jax import lax
from jax.experimental import pallas as pl
from jax.experimental.pallas import tpu as pltpu
from jax.experimental.pallas import tpu_sc as plsc

B = 16384
F = 26
EMBD = 32
N = B * F

NC = 2
NS = 16
NW = NC * NS
CHUNK = 128
FT = 4
NBB = B // CHUNK
RAW = FT * NBB * 8
A_ROWS = 3 * NBB * 8
A_PER_W = A_ROWS // NW
B_PER_W = (RAW - A_ROWS) // NW
SUP = 4
N_SUP = 26
LANES = 16


def _raw_view(x):
    return (
        jnp.pad(x.T, ((0, 6), (0, 0)))
        .reshape(FT, 8, NBB, CHUNK)
        .transpose(0, 2, 1, 3)
        .reshape(RAW, CHUNK)
    )


def kernel(xi, xv, ca_emb_weight):
    xi_n = _raw_view(xi.astype(jnp.int32))
    xv_n = _raw_view(xv)

    mesh = plsc.VectorSubcoreMesh(core_axis_name="c", subcore_axis_name="s")

    @functools.partial(
        pl.kernel,
        out_type=jax.ShapeDtypeStruct((F, EMBD // 8, NBB, 8, CHUNK),
                                      jnp.float32),
        mesh=mesh,
        scratch_types=[
            pltpu.VMEM((A_PER_W + B_PER_W, CHUNK), jnp.int32),
            pltpu.VMEM((A_PER_W + B_PER_W, CHUNK), jnp.float32),
            pltpu.VMEM((SUP, CHUNK, EMBD), jnp.float32),
            pltpu.VMEM((SUP, CHUNK, EMBD), jnp.float32),
            pltpu.VMEM((EMBD // 8, SUP, 8, CHUNK), jnp.float32),
            pltpu.VMEM((EMBD // 8, SUP, 8, CHUNK), jnp.float32),
            pltpu.SemaphoreType.DMA((2,)),
            pltpu.SemaphoreType.DMA((2,)),
        ],
        compiler_params=pltpu.CompilerParams(
            use_tc_tiling_on_sc=False, needs_layout_passes=False
        ),
    )
    def k(table_hbm, idx_hbm, xv_hbm, out_hbm,
          idx_v, xv_v, gbuf0, gbuf1, obuf0, obuf1, gsem, wsem):
        gbuf = (gbuf0, gbuf1)
        obuf = (obuf0, obuf1)
        wid = lax.axis_index("s") * NC + lax.axis_index("c")

        pltpu.sync_copy(idx_hbm.at[pl.ds(wid * A_PER_W, A_PER_W)],
                        idx_v.at[pl.ds(0, A_PER_W)])
        pltpu.sync_copy(idx_hbm.at[pl.ds(A_ROWS + wid * B_PER_W, B_PER_W)],
                        idx_v.at[pl.ds(A_PER_W, B_PER_W)])
        pltpu.sync_copy(xv_hbm.at[pl.ds(wid * A_PER_W, A_PER_W)],
                        xv_v.at[pl.ds(0, A_PER_W)])
        pltpu.sync_copy(xv_hbm.at[pl.ds(A_ROWS + wid * B_PER_W, B_PER_W)],
                        xv_v.at[pl.ds(A_PER_W, B_PER_W)])

        iota16 = lax.iota(jnp.int32, LANES)
        jsplat = [jnp.full((LANES,), j, jnp.int32) for j in range(SUP)]
        esplat = [jnp.full((LANES,), e, jnp.int32) for e in range(EMBD)]

        def sup_info(i):
            is_b = i >= 24
            fs = jnp.where(is_b, i - 24, lax.bitwise_and(i, 7))
            g = lax.shift_right_logical(i, 3)
            lb = jnp.where(is_b, A_PER_W, g * 32) + fs
            rr = jnp.where(is_b, A_ROWS + wid * B_PER_W,
                           wid * A_PER_W + g * 32) + fs
            f = lax.shift_right_logical(rr, 10) * 8 + fs
            bb0 = lax.shift_right_logical(lax.bitwise_and(rr, 1023), 3)
            return lb, f, bb0

        def start_gathers(i, b):
            lb, _, _ = sup_info(i)
            for j in range(SUP):
                pltpu.async_copy(
                    table_hbm.at[idx_v.at[lb + 8 * j]],
                    gbuf[b].at[j],
                    gsem.at[b],
                )

        def wait_gathers(i, b):
            lb, _, _ = sup_info(i)
            for j in range(SUP):
                pltpu.make_async_copy(
                    table_hbm.at[idx_v.at[lb + 8 * j]],
                    gbuf[b].at[j],
                    gsem.at[b],
                ).wait()

        def out_slice(i, tr):
            _, f, bb0 = sup_info(i)
            return out_hbm.at[f, tr, pl.ds(bb0, SUP)]

        def start_writebacks(i, b):
            for tr in range(EMBD // 8):
                pltpu.async_copy(obuf[b].at[tr], out_slice(i, tr), wsem.at[b])

        def wait_writebacks(i, b):
            for tr in range(EMBD // 8):
                pltpu.make_async_copy(
                    obuf[b].at[tr], out_slice(i, tr), wsem.at[b]
                ).wait()

        def compute(i, b):
            g_ref, o_ref = gbuf[b], obuf[b]
            lb, _, _ = sup_info(i)

            @plsc.parallel_loop(0, CHUNK // LANES, unroll=2)
            def _(bl0):
                lane0 = bl0 * LANES
                row_idx = lane0 + iota16
                xvv = [
                    xv_v[lb + 8 * j, pl.ds(lane0, LANES)] for j in range(SUP)
                ]
                for e in range(EMBD):
                    g = [
                        plsc.load_gather(g_ref, [jsplat[j], row_idx, esplat[e]])
                        for j in range(SUP)
                    ]
                    for j in range(SUP):
                        o_ref.at[e // 8, j, e % 8, pl.ds(lane0, LANES)][...] = (
                            g[j] * xvv[j]
                        )

        start_gathers(0, 0)

        @pl.loop(0, N_SUP, step=2)
        def _(i0):
            for b in range(2):
                i = i0 + b
                wait_gathers(i, b)
                @pl.when(i + 1 < N_SUP)
                def _():
                    start_gathers(i + 1, 1 - b)
                @pl.when(i >= 2)
                def _():
                    wait_writebacks(i - 2, b)
                compute(i, b)
                start_writebacks(i, b)

        for b in range(2):
            wait_writebacks(N_SUP - 2 + b, b)

    out5d = k(ca_emb_weight, xi_n, xv_n)
    return jnp.transpose(out5d, (2, 4, 0, 1, 3)).reshape(B, F, EMBD)


# speedup vs baseline: 1.1780x; 1.1081x over previous
import functools

import jax
import jax.numpy as jnp
from jax import lax
from jax.experimental import pallas as pl
from jax.experimental.pallas import tpu as pltpu
from jax.experimental.pallas import tpu_sc as plsc

B = 16384
F = 26
EMBD = 32
N = B * F

NC = 2
NS = 16
NW = NC * NS
CHUNK = 128
FT = 4
NBB = B // CHUNK
RAW = FT * NBB * 8
A_ROWS = 3 * NBB * 8
A_PER_W = A_ROWS // NW
B_PER_W = (RAW - A_ROWS) // NW
SUP = 4
N_SUP = 26
LANES = 16


def _raw_view(x):
    return (
        jnp.pad(x.T, ((0, 6), (0, 0)))
        .reshape(FT, 8, NBB, CHUNK)
        .transpose(0, 2, 1, 3)
        .reshape(RAW, CHUNK)
    )


TW = 2048
TGRID = (1000001 + TW - 1) // TW
VPAD = TGRID * TW


def _permuted_table(w):
    @functools.partial(
        pl.pallas_call,
        grid=(TGRID,),
        in_specs=[pl.BlockSpec((EMBD, TW), lambda g: (0, g))],
        out_specs=pl.BlockSpec((TW // 4, 128), lambda g: (g, 0)),
        out_shape=jax.ShapeDtypeStruct((VPAD // 4, 128), jnp.float32),
    )
    def tkern(in_ref, out_ref):
        x = in_ref[...]
        out_ref[...] = jnp.concatenate(
            [x[:, s * 512:(s + 1) * 512].T for s in range(4)], axis=1
        )

    return tkern(w.T).reshape(VPAD, EMBD)


def kernel(xi, xv, ca_emb_weight):
    xi_n = _raw_view(xi.astype(jnp.int32))
    xv_n = _raw_view(xv)
    table_rm = _permuted_table(ca_emb_weight)

    mesh = plsc.VectorSubcoreMesh(core_axis_name="c", subcore_axis_name="s")

    @functools.partial(
        pl.kernel,
        out_type=jax.ShapeDtypeStruct((F, EMBD // 8, NBB, 8, CHUNK),
                                      jnp.float32),
        mesh=mesh,
        scratch_types=[
            pltpu.VMEM((A_PER_W + B_PER_W, CHUNK), jnp.int32),
            pltpu.VMEM((A_PER_W + B_PER_W, CHUNK), jnp.float32),
            pltpu.VMEM((SUP, CHUNK, EMBD), jnp.float32),
            pltpu.VMEM((SUP, CHUNK, EMBD), jnp.float32),
            pltpu.VMEM((EMBD // 8, SUP, 8, CHUNK), jnp.float32),
            pltpu.VMEM((EMBD // 8, SUP, 8, CHUNK), jnp.float32),
            pltpu.SemaphoreType.DMA((2,)),
            pltpu.SemaphoreType.DMA((2,)),
        ],
        compiler_params=pltpu.CompilerParams(
            use_tc_tiling_on_sc=False, needs_layout_passes=False
        ),
    )
    def k(table_hbm, idx_hbm, xv_hbm, out_hbm,
          idx_v, xv_v, gbuf0, gbuf1, obuf0, obuf1, gsem, wsem):
        gbuf = (gbuf0, gbuf1)
        obuf = (obuf0, obuf1)
        wid = lax.axis_index("s") * NC + lax.axis_index("c")

        pltpu.sync_copy(idx_hbm.at[pl.ds(wid * A_PER_W, A_PER_W)],
                        idx_v.at[pl.ds(0, A_PER_W)])
        pltpu.sync_copy(idx_hbm.at[pl.ds(A_ROWS + wid * B_PER_W, B_PER_W)],
                        idx_v.at[pl.ds(A_PER_W, B_PER_W)])
        pltpu.sync_copy(xv_hbm.at[pl.ds(wid * A_PER_W, A_PER_W)],
                        xv_v.at[pl.ds(0, A_PER_W)])
        pltpu.sync_copy(xv_hbm.at[pl.ds(A_ROWS + wid * B_PER_W, B_PER_W)],
                        xv_v.at[pl.ds(A_PER_W, B_PER_W)])

        @plsc.parallel_loop(0, A_PER_W + B_PER_W)
        def _(r):
            for q in range(CHUNK // LANES):
                sl = (r, pl.ds(q * LANES, LANES))
                v = idx_v[sl]
                h = (
                    lax.shift_left(lax.shift_right_logical(v, 11), 11)
                    | lax.shift_left(lax.bitwise_and(v, 511), 2)
                    | lax.bitwise_and(lax.shift_right_logical(v, 9), 3)
                )
                idx_v.at[sl][...] = h

        iota16 = lax.iota(jnp.int32, LANES)
        jsplat = [jnp.full((LANES,), j, jnp.int32) for j in range(SUP)]
        esplat = [jnp.full((LANES,), e, jnp.int32) for e in range(EMBD)]

        def sup_info(i):
            is_b = i >= 24
            fs = jnp.where(is_b, i - 24, lax.bitwise_and(i, 7))
            g = lax.shift_right_logical(i, 3)
            lb = jnp.where(is_b, A_PER_W, g * 32) + fs
            rr = jnp.where(is_b, A_ROWS + wid * B_PER_W,
                           wid * A_PER_W + g * 32) + fs
            f = lax.shift_right_logical(rr, 10) * 8 + fs
            bb0 = lax.shift_right_logical(lax.bitwise_and(rr, 1023), 3)
            return lb, f, bb0

        def start_gathers(i, b):
            lb, _, _ = sup_info(i)
            for j in range(SUP):
                pltpu.async_copy(
                    table_hbm.at[idx_v.at[lb + 8 * j]],
                    gbuf[b].at[j],
                    gsem.at[b],
                )

        def wait_gathers(i, b):
            lb, _, _ = sup_info(i)
            for j in range(SUP):
                pltpu.make_async_copy(
                    table_hbm.at[idx_v.at[lb + 8 * j]],
                    gbuf[b].at[j],
                    gsem.at[b],
                ).wait()

        def out_slice(i, tr):
            _, f, bb0 = sup_info(i)
            return out_hbm.at[f, tr, pl.ds(bb0, SUP)]

        def start_writebacks(i, b):
            for tr in range(EMBD // 8):
                pltpu.async_copy(obuf[b].at[tr], out_slice(i, tr), wsem.at[b])

        def wait_writebacks(i, b):
            for tr in range(EMBD // 8):
                pltpu.make_async_copy(
                    obuf[b].at[tr], out_slice(i, tr), wsem.at[b]
                ).wait()

        def compute(i, b):
            g_ref, o_ref = gbuf[b], obuf[b]
            lb, _, _ = sup_info(i)

            @plsc.parallel_loop(0, CHUNK // LANES, unroll=2)
            def _(bl0):
                lane0 = bl0 * LANES
                row_idx = lane0 + iota16
                xvv = [
                    xv_v[lb + 8 * j, pl.ds(lane0, LANES)] for j in range(SUP)
                ]
                for e in range(EMBD):
                    g = [
                        plsc.load_gather(g_ref, [jsplat[j], row_idx, esplat[e]])
                        for j in range(SUP)
                    ]
                    for j in range(SUP):
                        o_ref.at[e // 8, j, e % 8, pl.ds(lane0, LANES)][...] = (
                            g[j] * xvv[j]
                        )

        start_gathers(0, 0)

        @pl.loop(0, N_SUP, step=2)
        def _(i0):
            for b in range(2):
                i = i0 + b
                wait_gathers(i, b)
                @pl.when(i + 1 < N_SUP)
                def _():
                    start_gathers(i + 1, 1 - b)
                @pl.when(i >= 2)
                def _():
                    wait_writebacks(i - 2, b)
                compute(i, b)
                start_writebacks(i, b)

        for b in range(2):
            wait_writebacks(N_SUP - 2 + b, b)

    out5d = k(table_rm, xi_n, xv_n)
    return jnp.transpose(out5d, (2, 4, 0, 1, 3)).reshape(B, F, EMBD)


# speedup vs baseline: 1.5512x; 1.3168x over previous
import functools

import jax
import jax.numpy as jnp
from jax import lax
from jax.experimental import pallas as pl
from jax.experimental.pallas import tpu as pltpu
from jax.experimental.pallas import tpu_sc as plsc

B = 16384
F = 26
EMBD = 32
N = B * F

NC = 2
NS = 16
NW = NC * NS
CHUNK = 128
FT = 4
NBB = B // CHUNK
RAW = FT * NBB * 8
A_ROWS = 3 * NBB * 8
A_PER_W = A_ROWS // NW
B_PER_W = (RAW - A_ROWS) // NW
SUP = 4
N_SUP = 26
LANES = 16


def _raw_view(x):
    return (
        jnp.pad(x.T, ((0, 6), (0, 0)))
        .reshape(FT, 8, NBB, CHUNK)
        .transpose(0, 2, 1, 3)
        .reshape(RAW, CHUNK)
    )


TW = 8192
TGRID = (1000001 + TW - 1) // TW
VPAD = TGRID * TW


def _permuted_table(w):
    @functools.partial(
        pl.pallas_call,
        grid=(TGRID,),
        in_specs=[pl.BlockSpec((EMBD, TW), lambda g: (0, g))],
        out_specs=pl.BlockSpec((TW // 4, 128), lambda g: (g, 0)),
        out_shape=jax.ShapeDtypeStruct((VPAD // 4, 128), jnp.float32),
    )
    def tkern(in_ref, out_ref):
        x = in_ref[...]
        q = TW // 4
        out_ref[...] = jnp.concatenate(
            [x[:, s * q:(s + 1) * q].T for s in range(4)], axis=1
        )

    return tkern(w.T).reshape(VPAD, EMBD)


def kernel(xi, xv, ca_emb_weight):
    xi_n = _raw_view(xi.astype(jnp.int32))
    xv_n = _raw_view(xv)
    table_rm = _permuted_table(ca_emb_weight)

    mesh = plsc.VectorSubcoreMesh(core_axis_name="c", subcore_axis_name="s")

    @functools.partial(
        pl.kernel,
        out_type=jax.ShapeDtypeStruct((F, EMBD // 8, NBB, 8, CHUNK),
                                      jnp.float32),
        mesh=mesh,
        scratch_types=[
            pltpu.VMEM((A_PER_W + B_PER_W, CHUNK), jnp.int32),
            pltpu.VMEM((A_PER_W + B_PER_W, CHUNK), jnp.float32),
            pltpu.VMEM((SUP, CHUNK, EMBD), jnp.float32),
            pltpu.VMEM((SUP, CHUNK, EMBD), jnp.float32),
            pltpu.VMEM((EMBD // 8, SUP, 8, CHUNK), jnp.float32),
            pltpu.VMEM((EMBD // 8, SUP, 8, CHUNK), jnp.float32),
            pltpu.SemaphoreType.DMA((2,)),
            pltpu.SemaphoreType.DMA((2,)),
        ],
        compiler_params=pltpu.CompilerParams(
            use_tc_tiling_on_sc=False, needs_layout_passes=False
        ),
    )
    def k(table_hbm, idx_hbm, xv_hbm, out_hbm,
          idx_v, xv_v, gbuf0, gbuf1, obuf0, obuf1, gsem, wsem):
        gbuf = (gbuf0, gbuf1)
        obuf = (obuf0, obuf1)
        wid = lax.axis_index("s") * NC + lax.axis_index("c")

        pltpu.sync_copy(idx_hbm.at[pl.ds(wid * A_PER_W, A_PER_W)],
                        idx_v.at[pl.ds(0, A_PER_W)])
        pltpu.sync_copy(idx_hbm.at[pl.ds(A_ROWS + wid * B_PER_W, B_PER_W)],
                        idx_v.at[pl.ds(A_PER_W, B_PER_W)])
        pltpu.sync_copy(xv_hbm.at[pl.ds(wid * A_PER_W, A_PER_W)],
                        xv_v.at[pl.ds(0, A_PER_W)])
        pltpu.sync_copy(xv_hbm.at[pl.ds(A_ROWS + wid * B_PER_W, B_PER_W)],
                        xv_v.at[pl.ds(A_PER_W, B_PER_W)])

        @plsc.parallel_loop(0, A_PER_W + B_PER_W)
        def _(r):
            for q in range(CHUNK // LANES):
                sl = (r, pl.ds(q * LANES, LANES))
                v = idx_v[sl]
                lg = TW.bit_length() - 1
                h = (
                    lax.shift_left(lax.shift_right_logical(v, lg), lg)
                    | lax.shift_left(lax.bitwise_and(v, TW // 4 - 1), 2)
                    | lax.bitwise_and(lax.shift_right_logical(v, lg - 2), 3)
                )
                idx_v.at[sl][...] = h

        iota16 = lax.iota(jnp.int32, LANES)
        jsplat = [jnp.full((LANES,), j, jnp.int32) for j in range(SUP)]
        esplat = [jnp.full((LANES,), e, jnp.int32) for e in range(EMBD)]

        def sup_info(i):
            is_b = i >= 24
            fs = jnp.where(is_b, i - 24, lax.bitwise_and(i, 7))
            g = lax.shift_right_logical(i, 3)
            lb = jnp.where(is_b, A_PER_W, g * 32) + fs
            rr = jnp.where(is_b, A_ROWS + wid * B_PER_W,
                           wid * A_PER_W + g * 32) + fs
            f = lax.shift_right_logical(rr, 10) * 8 + fs
            bb0 = lax.shift_right_logical(lax.bitwise_and(rr, 1023), 3)
            return lb, f, bb0

        def start_gathers(i, b):
            lb, _, _ = sup_info(i)
            for j in range(SUP):
                pltpu.async_copy(
                    table_hbm.at[idx_v.at[lb + 8 * j]],
                    gbuf[b].at[j],
                    gsem.at[b],
                )

        def wait_gathers(i, b):
            lb, _, _ = sup_info(i)
            for j in range(SUP):
                pltpu.make_async_copy(
                    table_hbm.at[idx_v.at[lb + 8 * j]],
                    gbuf[b].at[j],
                    gsem.at[b],
                ).wait()

        def out_slice(i, tr):
            _, f, bb0 = sup_info(i)
            return out_hbm.at[f, tr, pl.ds(bb0, SUP)]

        def start_writebacks(i, b):
            for tr in range(EMBD // 8):
                pltpu.async_copy(obuf[b].at[tr], out_slice(i, tr), wsem.at[b])

        def wait_writebacks(i, b):
            for tr in range(EMBD // 8):
                pltpu.make_async_copy(
                    obuf[b].at[tr], out_slice(i, tr), wsem.at[b]
                ).wait()

        def compute(i, b):
            g_ref, o_ref = gbuf[b], obuf[b]
            lb, _, _ = sup_info(i)

            @plsc.parallel_loop(0, CHUNK // LANES, unroll=2)
            def _(bl0):
                lane0 = bl0 * LANES
                row_idx = lane0 + iota16
                xvv = [
                    xv_v[lb + 8 * j, pl.ds(lane0, LANES)] for j in range(SUP)
                ]
                for e in range(EMBD):
                    g = [
                        plsc.load_gather(g_ref, [jsplat[j], row_idx, esplat[e]])
                        for j in range(SUP)
                    ]
                    for j in range(SUP):
                        o_ref.at[e // 8, j, e % 8, pl.ds(lane0, LANES)][...] = (
                            g[j] * xvv[j]
                        )

        start_gathers(0, 0)

        @pl.loop(0, N_SUP, step=2)
        def _(i0):
            for b in range(2):
                i = i0 + b
                wait_gathers(i, b)
                @pl.when(i + 1 < N_SUP)
                def _():
                    start_gathers(i + 1, 1 - b)
                @pl.when(i >= 2)
                def _():
                    wait_writebacks(i - 2, b)
                compute(i, b)
                start_writebacks(i, b)

        for b in range(2):
            wait_writebacks(N_SUP - 2 + b, b)

    out5d = k(table_rm, xi_n, xv_n)
    return jnp.transpose(out5d, (2, 4, 0, 1, 3)).reshape(B, F, EMBD)


# speedup vs baseline: 1.5627x; 1.0074x over previous
import functools

import jax
import jax.numpy as jnp
from jax import lax
from jax.experimental import pallas as pl
from jax.experimental.pallas import tpu as pltpu
from jax.experimental.pallas import tpu_sc as plsc

B = 16384
F = 26
EMBD = 32
N = B * F

NC = 2
NS = 16
NW = NC * NS
CHUNK = 128
FT = 4
NBB = B // CHUNK
RAW = FT * NBB * 8
A_ROWS = 3 * NBB * 8
A_PER_W = A_ROWS // NW
B_PER_W = (RAW - A_ROWS) // NW
SUP = 4
N_SUP = 26
LANES = 16


def _raw_view(x):
    return (
        jnp.pad(x.T, ((0, 6), (0, 0)))
        .reshape(FT, 8, NBB, CHUNK)
        .transpose(0, 2, 1, 3)
        .reshape(RAW, CHUNK)
    )


TW = 32768
TGRID = (1000001 + TW - 1) // TW
VPAD = TGRID * TW


def _permuted_table(w):
    @functools.partial(
        pl.pallas_call,
        grid=(TGRID,),
        in_specs=[pl.BlockSpec((EMBD, TW), lambda g: (0, g))],
        out_specs=pl.BlockSpec((TW // 4, 128), lambda g: (g, 0)),
        out_shape=jax.ShapeDtypeStruct((VPAD // 4, 128), jnp.float32),
    )
    def tkern(in_ref, out_ref):
        x = in_ref[...]
        q = TW // 4
        out_ref[...] = jnp.concatenate(
            [x[:, s * q:(s + 1) * q].T for s in range(4)], axis=1
        )

    return tkern(w.T).reshape(VPAD, EMBD)


def kernel(xi, xv, ca_emb_weight):
    xi_n = _raw_view(xi.astype(jnp.int32))
    xv_n = _raw_view(xv)
    table_rm = _permuted_table(ca_emb_weight)

    mesh = plsc.VectorSubcoreMesh(core_axis_name="c", subcore_axis_name="s")

    @functools.partial(
        pl.kernel,
        out_type=jax.ShapeDtypeStruct((F, EMBD // 8, NBB, 8, CHUNK),
                                      jnp.float32),
        mesh=mesh,
        scratch_types=[
            pltpu.VMEM((A_PER_W + B_PER_W, CHUNK), jnp.int32),
            pltpu.VMEM((A_PER_W + B_PER_W, CHUNK), jnp.float32),
            pltpu.VMEM((SUP, CHUNK, EMBD), jnp.float32),
            pltpu.VMEM((SUP, CHUNK, EMBD), jnp.float32),
            pltpu.VMEM((EMBD // 8, SUP, 8, CHUNK), jnp.float32),
            pltpu.VMEM((EMBD // 8, SUP, 8, CHUNK), jnp.float32),
            pltpu.SemaphoreType.DMA((2,)),
            pltpu.SemaphoreType.DMA((2,)),
        ],
        compiler_params=pltpu.CompilerParams(
            use_tc_tiling_on_sc=False, needs_layout_passes=False
        ),
    )
    def k(table_hbm, idx_hbm, xv_hbm, out_hbm,
          idx_v, xv_v, gbuf0, gbuf1, obuf0, obuf1, gsem, wsem):
        gbuf = (gbuf0, gbuf1)
        obuf = (obuf0, obuf1)
        wid = lax.axis_index("s") * NC + lax.axis_index("c")

        pltpu.sync_copy(idx_hbm.at[pl.ds(wid * A_PER_W, A_PER_W)],
                        idx_v.at[pl.ds(0, A_PER_W)])
        pltpu.sync_copy(idx_hbm.at[pl.ds(A_ROWS + wid * B_PER_W, B_PER_W)],
                        idx_v.at[pl.ds(A_PER_W, B_PER_W)])
        pltpu.sync_copy(xv_hbm.at[pl.ds(wid * A_PER_W, A_PER_W)],
                        xv_v.at[pl.ds(0, A_PER_W)])
        pltpu.sync_copy(xv_hbm.at[pl.ds(A_ROWS + wid * B_PER_W, B_PER_W)],
                        xv_v.at[pl.ds(A_PER_W, B_PER_W)])

        @plsc.parallel_loop(0, A_PER_W + B_PER_W)
        def _(r):
            for q in range(CHUNK // LANES):
                sl = (r, pl.ds(q * LANES, LANES))
                v = idx_v[sl]
                lg = TW.bit_length() - 1
                h = (
                    lax.shift_left(lax.shift_right_logical(v, lg), lg)
                    | lax.shift_left(lax.bitwise_and(v, TW // 4 - 1), 2)
                    | lax.bitwise_and(lax.shift_right_logical(v, lg - 2), 3)
                )
                idx_v.at[sl][...] = h

        iota16 = lax.iota(jnp.int32, LANES)
        jsplat = [jnp.full((LANES,), j, jnp.int32) for j in range(SUP)]
        esplat = [jnp.full((LANES,), e, jnp.int32) for e in range(EMBD)]

        def sup_info(i):
            is_b = i >= 24
            fs = jnp.where(is_b, i - 24, lax.bitwise_and(i, 7))
            g = lax.shift_right_logical(i, 3)
            lb = jnp.where(is_b, A_PER_W, g * 32) + fs
            rr = jnp.where(is_b, A_ROWS + wid * B_PER_W,
                           wid * A_PER_W + g * 32) + fs
            f = lax.shift_right_logical(rr, 10) * 8 + fs
            bb0 = lax.shift_right_logical(lax.bitwise_and(rr, 1023), 3)
            return lb, f, bb0

        def start_gathers(i, b):
            lb, _, _ = sup_info(i)
            for j in range(SUP):
                pltpu.async_copy(
                    table_hbm.at[idx_v.at[lb + 8 * j]],
                    gbuf[b].at[j],
                    gsem.at[b],
                )

        def wait_gathers(i, b):
            lb, _, _ = sup_info(i)
            for j in range(SUP):
                pltpu.make_async_copy(
                    table_hbm.at[idx_v.at[lb + 8 * j]],
                    gbuf[b].at[j],
                    gsem.at[b],
                ).wait()

        def out_slice(i, tr):
            _, f, bb0 = sup_info(i)
            return out_hbm.at[f, tr, pl.ds(bb0, SUP)]

        def start_writebacks(i, b):
            for tr in range(EMBD // 8):
                pltpu.async_copy(obuf[b].at[tr], out_slice(i, tr), wsem.at[b])

        def wait_writebacks(i, b):
            for tr in range(EMBD // 8):
                pltpu.make_async_copy(
                    obuf[b].at[tr], out_slice(i, tr), wsem.at[b]
                ).wait()

        def compute(i, b):
            g_ref, o_ref = gbuf[b], obuf[b]
            lb, _, _ = sup_info(i)

            @plsc.parallel_loop(0, CHUNK // LANES, unroll=2)
            def _(bl0):
                lane0 = bl0 * LANES
                row_idx = lane0 + iota16
                xvv = [
                    xv_v[lb + 8 * j, pl.ds(lane0, LANES)] for j in range(SUP)
                ]
                for e in range(EMBD):
                    g = [
                        plsc.load_gather(g_ref, [jsplat[j], row_idx, esplat[e]])
                        for j in range(SUP)
                    ]
                    for j in range(SUP):
                        o_ref.at[e // 8, j, e % 8, pl.ds(lane0, LANES)][...] = (
                            g[j] * xvv[j]
                        )

        start_gathers(0, 0)

        @pl.loop(0, N_SUP, step=2)
        def _(i0):
            for b in range(2):
                i = i0 + b
                wait_gathers(i, b)
                @pl.when(i + 1 < N_SUP)
                def _():
                    start_gathers(i + 1, 1 - b)
                @pl.when(i >= 2)
                def _():
                    wait_writebacks(i - 2, b)
                compute(i, b)
                start_writebacks(i, b)

        for b in range(2):
            wait_writebacks(N_SUP - 2 + b, b)

    out5d = k(table_rm, xi_n, xv_n)
    return jnp.transpose(out5d, (2, 4, 0, 1, 3)).reshape(B, F, EMBD)


# speedup vs baseline: 1.6078x; 1.0288x over previous
import functools

import jax
import jax.numpy as jnp
from jax import lax
from jax.experimental import pallas as pl
from jax.experimental.pallas import tpu as pltpu
from jax.experimental.pallas import tpu_sc as plsc

B = 16384
F = 26
EMBD = 32
N = B * F

NC = 2
NS = 16
NW = NC * NS
CHUNK = 128
FT = 4
NBB = B // CHUNK
RAW = FT * NBB * 8
A_ROWS = 3 * NBB * 8
A_PER_W = A_ROWS // NW
B_PER_W = (RAW - A_ROWS) // NW
SUP = 2
N_SUP = 52
NBUF = 4
LANES = 16


def _raw_view(x):
    return (
        jnp.pad(x.T, ((0, 6), (0, 0)))
        .reshape(FT, 8, NBB, CHUNK)
        .transpose(0, 2, 1, 3)
        .reshape(RAW, CHUNK)
    )


TW = 32768
TGRID = (1000001 + TW - 1) // TW
VPAD = TGRID * TW


def _permuted_table(w):
    @functools.partial(
        pl.pallas_call,
        grid=(TGRID,),
        in_specs=[pl.BlockSpec((EMBD, TW), lambda g: (0, g))],
        out_specs=pl.BlockSpec((TW // 4, 128), lambda g: (g, 0)),
        out_shape=jax.ShapeDtypeStruct((VPAD // 4, 128), jnp.float32),
    )
    def tkern(in_ref, out_ref):
        x = in_ref[...]
        q = TW // 4
        out_ref[...] = jnp.concatenate(
            [x[:, s * q:(s + 1) * q].T for s in range(4)], axis=1
        )

    return tkern(w.T).reshape(VPAD, EMBD)


def kernel(xi, xv, ca_emb_weight):
    xi_n = _raw_view(xi.astype(jnp.int32))
    xv_n = _raw_view(xv)
    table_rm = _permuted_table(ca_emb_weight)

    mesh = plsc.VectorSubcoreMesh(core_axis_name="c", subcore_axis_name="s")

    @functools.partial(
        pl.kernel,
        out_type=jax.ShapeDtypeStruct((F, EMBD // 8, NBB, 8, CHUNK),
                                      jnp.float32),
        mesh=mesh,
        scratch_types=[
            pltpu.VMEM((A_PER_W + B_PER_W, CHUNK), jnp.int32),
            pltpu.VMEM((A_PER_W + B_PER_W, CHUNK), jnp.float32),
        ]
        + [pltpu.VMEM((SUP, CHUNK, EMBD), jnp.float32) for _ in range(NBUF)]
        + [pltpu.VMEM((EMBD // 8, SUP, 8, CHUNK), jnp.float32)
           for _ in range(NBUF)]
        + [
            pltpu.SemaphoreType.DMA((NBUF,)),
            pltpu.SemaphoreType.DMA((NBUF,)),
        ],
        compiler_params=pltpu.CompilerParams(
            use_tc_tiling_on_sc=False, needs_layout_passes=False
        ),
    )
    def k(table_hbm, idx_hbm, xv_hbm, out_hbm, idx_v, xv_v, *rest):
        gbuf = rest[:NBUF]
        obuf = rest[NBUF:2 * NBUF]
        gsem, wsem = rest[2 * NBUF], rest[2 * NBUF + 1]
        wid = lax.axis_index("s") * NC + lax.axis_index("c")

        pltpu.sync_copy(idx_hbm.at[pl.ds(wid * A_PER_W, A_PER_W)],
                        idx_v.at[pl.ds(0, A_PER_W)])
        pltpu.sync_copy(idx_hbm.at[pl.ds(A_ROWS + wid * B_PER_W, B_PER_W)],
                        idx_v.at[pl.ds(A_PER_W, B_PER_W)])
        pltpu.sync_copy(xv_hbm.at[pl.ds(wid * A_PER_W, A_PER_W)],
                        xv_v.at[pl.ds(0, A_PER_W)])
        pltpu.sync_copy(xv_hbm.at[pl.ds(A_ROWS + wid * B_PER_W, B_PER_W)],
                        xv_v.at[pl.ds(A_PER_W, B_PER_W)])

        @plsc.parallel_loop(0, A_PER_W + B_PER_W)
        def _(r):
            for q in range(CHUNK // LANES):
                sl = (r, pl.ds(q * LANES, LANES))
                v = idx_v[sl]
                lg = TW.bit_length() - 1
                h = (
                    lax.shift_left(lax.shift_right_logical(v, lg), lg)
                    | lax.shift_left(lax.bitwise_and(v, TW // 4 - 1), 2)
                    | lax.bitwise_and(lax.shift_right_logical(v, lg - 2), 3)
                )
                idx_v.at[sl][...] = h

        iota16 = lax.iota(jnp.int32, LANES)
        jsplat = [jnp.full((LANES,), j, jnp.int32) for j in range(SUP)]
        esplat = [jnp.full((LANES,), e, jnp.int32) for e in range(EMBD)]

        def sup_info(i):
            is_b = i >= 48
            t = i - 48
            fs = jnp.where(is_b, lax.bitwise_and(t, 1), lax.bitwise_and(i, 7))
            grp16 = jnp.where(is_b, lax.shift_right_logical(t, 1),
                              lax.shift_right_logical(i, 3))
            lb = jnp.where(is_b, A_PER_W, 0) + grp16 * 16 + fs
            rr = jnp.where(is_b, A_ROWS + wid * B_PER_W,
                           wid * A_PER_W) + grp16 * 16 + fs
            f = lax.shift_right_logical(rr, 10) * 8 + fs
            bb0 = lax.shift_right_logical(lax.bitwise_and(rr, 1023), 3)
            return lb, f, bb0

        def start_gathers(i, b):
            lb, _, _ = sup_info(i)
            for j in range(SUP):
                pltpu.async_copy(
                    table_hbm.at[idx_v.at[lb + 8 * j]],
                    gbuf[b].at[j],
                    gsem.at[b],
                )

        def wait_gathers(i, b):
            lb, _, _ = sup_info(i)
            for j in range(SUP):
                pltpu.make_async_copy(
                    table_hbm.at[idx_v.at[lb + 8 * j]],
                    gbuf[b].at[j],
                    gsem.at[b],
                ).wait()

        def out_slice(i, tr):
            _, f, bb0 = sup_info(i)
            return out_hbm.at[f, tr, pl.ds(bb0, SUP)]

        def start_writebacks(i, b):
            for tr in range(EMBD // 8):
                pltpu.async_copy(obuf[b].at[tr], out_slice(i, tr), wsem.at[b])

        def wait_writebacks(i, b):
            for tr in range(EMBD // 8):
                pltpu.make_async_copy(
                    obuf[b].at[tr], out_slice(i, tr), wsem.at[b]
                ).wait()

        def compute(i, b):
            g_ref, o_ref = gbuf[b], obuf[b]
            lb, _, _ = sup_info(i)

            @plsc.parallel_loop(0, CHUNK // LANES, unroll=2)
            def _(bl0):
                lane0 = bl0 * LANES
                row_idx = lane0 + iota16
                xvv = [
                    xv_v[lb + 8 * j, pl.ds(lane0, LANES)] for j in range(SUP)
                ]
                for e in range(EMBD):
                    g = [
                        plsc.load_gather(g_ref, [jsplat[j], row_idx, esplat[e]])
                        for j in range(SUP)
                    ]
                    for j in range(SUP):
                        o_ref.at[e // 8, j, e % 8, pl.ds(lane0, LANES)][...] = (
                            g[j] * xvv[j]
                        )

        for i in range(NBUF - 1):
            start_gathers(i, i)

        @pl.loop(0, N_SUP, step=NBUF)
        def _(i0):
            for b in range(NBUF):
                i = i0 + b
                wait_gathers(i, b)
                @pl.when(i + NBUF - 1 < N_SUP)
                def _():
                    start_gathers(i + NBUF - 1, (b + NBUF - 1) % NBUF)
                @pl.when(i >= NBUF)
                def _():
                    wait_writebacks(i - NBUF, b)
                compute(i, b)
                start_writebacks(i, b)

        for b in range(NBUF):
            wait_writebacks(N_SUP - NBUF + b, b)

    out5d = k(table_rm, xi_n, xv_n)
    return jnp.transpose(out5d, (2, 4, 0, 1, 3)).reshape(B, F, EMBD)


# speedup vs baseline: 2.0585x; 1.2803x over previous
import functools

import jax
import jax.numpy as jnp
from jax import lax
from jax.experimental import pallas as pl
from jax.experimental.pallas import tpu as pltpu
from jax.experimental.pallas import tpu_sc as plsc

B = 16384
F = 26
EMBD = 32
N = B * F

NC = 2
NS = 16
NW = NC * NS
CHUNK = 128
FT = 4
NBB = B // CHUNK
RAW = FT * NBB * 8
A_ROWS = 3 * NBB * 8
A_PER_W = A_ROWS // NW
B_PER_W = (RAW - A_ROWS) // NW
SUP = 2
N_SUP = 52
NBUF = 4
LANES = 16


def _raw_view(x):
    return (
        jnp.pad(x.T, ((0, 6), (0, 0)))
        .reshape(FT, 8, NBB, CHUNK)
        .transpose(0, 2, 1, 3)
        .reshape(RAW, CHUNK)
    )


TW = 32768
TGRID = (1000001 + TW - 1) // TW
VPAD = TGRID * TW


def _permuted_table(w):
    @functools.partial(
        pl.pallas_call,
        grid=(TGRID,),
        in_specs=[pl.BlockSpec((EMBD, TW), lambda g: (0, g))],
        out_specs=pl.BlockSpec((TW // 4, 128), lambda g: (g, 0)),
        out_shape=jax.ShapeDtypeStruct((VPAD // 4, 128), jnp.float32),
    )
    def tkern(in_ref, out_ref):
        x = in_ref[...]
        q = TW // 4
        out_ref[...] = jnp.concatenate(
            [x[:, s * q:(s + 1) * q].T for s in range(4)], axis=1
        )

    return tkern(w.T).reshape(VPAD, EMBD)


def kernel(xi, xv, ca_emb_weight):
    xi_n = _raw_view(xi.astype(jnp.int32))
    xv_n = _raw_view(xv)
    table_rm = _permuted_table(ca_emb_weight)

    mesh = plsc.VectorSubcoreMesh(core_axis_name="c", subcore_axis_name="s")

    @functools.partial(
        pl.kernel,
        out_type=jax.ShapeDtypeStruct((F, EMBD // 8, NBB, 8, CHUNK),
                                      jnp.float32),
        mesh=mesh,
        scratch_types=[
            pltpu.VMEM((A_PER_W + B_PER_W, CHUNK), jnp.int32),
            pltpu.VMEM((A_PER_W + B_PER_W, CHUNK), jnp.float32),
        ]
        + [pltpu.VMEM((SUP, CHUNK, EMBD), jnp.float32) for _ in range(NBUF)]
        + [pltpu.VMEM((SUP, CHUNK, EMBD + 1), jnp.float32)]
        + [pltpu.VMEM((EMBD // 8, SUP, 8, CHUNK), jnp.float32)
           for _ in range(NBUF)]
        + [
            pltpu.SemaphoreType.DMA((NBUF,)),
            pltpu.SemaphoreType.DMA((NBUF,)),
        ],
        compiler_params=pltpu.CompilerParams(
            use_tc_tiling_on_sc=False, needs_layout_passes=False
        ),
    )
    def k(table_hbm, idx_hbm, xv_hbm, out_hbm, idx_v, xv_v, *rest):
        gbuf = rest[:NBUF]
        gpad = rest[NBUF]
        obuf = rest[NBUF + 1:NBUF + 1 + NBUF]
        gsem, wsem = rest[-2], rest[-1]
        wid = lax.axis_index("s") * NC + lax.axis_index("c")

        pltpu.sync_copy(idx_hbm.at[pl.ds(wid * A_PER_W, A_PER_W)],
                        idx_v.at[pl.ds(0, A_PER_W)])
        pltpu.sync_copy(idx_hbm.at[pl.ds(A_ROWS + wid * B_PER_W, B_PER_W)],
                        idx_v.at[pl.ds(A_PER_W, B_PER_W)])
        pltpu.sync_copy(xv_hbm.at[pl.ds(wid * A_PER_W, A_PER_W)],
                        xv_v.at[pl.ds(0, A_PER_W)])
        pltpu.sync_copy(xv_hbm.at[pl.ds(A_ROWS + wid * B_PER_W, B_PER_W)],
                        xv_v.at[pl.ds(A_PER_W, B_PER_W)])

        @plsc.parallel_loop(0, A_PER_W + B_PER_W)
        def _(r):
            for q in range(CHUNK // LANES):
                sl = (r, pl.ds(q * LANES, LANES))
                v = idx_v[sl]
                lg = TW.bit_length() - 1
                h = (
                    lax.shift_left(lax.shift_right_logical(v, lg), lg)
                    | lax.shift_left(lax.bitwise_and(v, TW // 4 - 1), 2)
                    | lax.bitwise_and(lax.shift_right_logical(v, lg - 2), 3)
                )
                idx_v.at[sl][...] = h

        iota16 = lax.iota(jnp.int32, LANES)
        jsplat = [jnp.full((LANES,), j, jnp.int32) for j in range(SUP)]
        esplat = [jnp.full((LANES,), e, jnp.int32) for e in range(EMBD)]

        def sup_info(i):
            is_b = i >= 48
            t = i - 48
            fs = jnp.where(is_b, lax.bitwise_and(t, 1), lax.bitwise_and(i, 7))
            grp16 = jnp.where(is_b, lax.shift_right_logical(t, 1),
                              lax.shift_right_logical(i, 3))
            lb = jnp.where(is_b, A_PER_W, 0) + grp16 * 16 + fs
            rr = jnp.where(is_b, A_ROWS + wid * B_PER_W,
                           wid * A_PER_W) + grp16 * 16 + fs
            f = lax.shift_right_logical(rr, 10) * 8 + fs
            bb0 = lax.shift_right_logical(lax.bitwise_and(rr, 1023), 3)
            return lb, f, bb0

        def start_gathers(i, b):
            lb, _, _ = sup_info(i)
            for j in range(SUP):
                pltpu.async_copy(
                    table_hbm.at[idx_v.at[lb + 8 * j]],
                    gbuf[b].at[j],
                    gsem.at[b],
                )

        def wait_gathers(i, b):
            lb, _, _ = sup_info(i)
            for j in range(SUP):
                pltpu.make_async_copy(
                    table_hbm.at[idx_v.at[lb + 8 * j]],
                    gbuf[b].at[j],
                    gsem.at[b],
                ).wait()

        def out_slice(i, tr):
            _, f, bb0 = sup_info(i)
            return out_hbm.at[f, tr, pl.ds(bb0, SUP)]

        def start_writebacks(i, b):
            for tr in range(EMBD // 8):
                pltpu.async_copy(obuf[b].at[tr], out_slice(i, tr), wsem.at[b])

        def wait_writebacks(i, b):
            for tr in range(EMBD // 8):
                pltpu.make_async_copy(
                    obuf[b].at[tr], out_slice(i, tr), wsem.at[b]
                ).wait()

        def compute(i, b):
            src = gbuf[b]

            @plsc.parallel_loop(0, CHUNK)
            def _(r):
                for j in range(SUP):
                    for hh in range(EMBD // LANES):
                        sl = pl.ds(hh * LANES, LANES)
                        gpad.at[j, r, sl][...] = src[j, r, sl]

            g_ref, o_ref = gpad, obuf[b]
            lb, _, _ = sup_info(i)

            @plsc.parallel_loop(0, CHUNK // LANES, unroll=2)
            def _(bl0):
                lane0 = bl0 * LANES
                row_idx = lane0 + iota16
                xvv = [
                    xv_v[lb + 8 * j, pl.ds(lane0, LANES)] for j in range(SUP)
                ]
                for e in range(EMBD):
                    g = [
                        plsc.load_gather(g_ref, [jsplat[j], row_idx, esplat[e]])
                        for j in range(SUP)
                    ]
                    for j in range(SUP):
                        o_ref.at[e // 8, j, e % 8, pl.ds(lane0, LANES)][...] = (
                            g[j] * xvv[j]
                        )

        for i in range(NBUF - 1):
            start_gathers(i, i)

        @pl.loop(0, N_SUP, step=NBUF)
        def _(i0):
            for b in range(NBUF):
                i = i0 + b
                wait_gathers(i, b)
                @pl.when(i + NBUF - 1 < N_SUP)
                def _():
                    start_gathers(i + NBUF - 1, (b + NBUF - 1) % NBUF)
                @pl.when(i >= NBUF)
                def _():
                    wait_writebacks(i - NBUF, b)
                compute(i, b)
                start_writebacks(i, b)

        for b in range(NBUF):
            wait_writebacks(N_SUP - NBUF + b, b)

    out5d = k(table_rm, xi_n, xv_n)
    return jnp.transpose(out5d, (2, 4, 0, 1, 3)).reshape(B, F, EMBD)


# speedup vs baseline: 2.0623x; 1.0018x over previous
import functools

import jax
import jax.numpy as jnp
from jax import lax
from jax.experimental import pallas as pl
from jax.experimental.pallas import tpu as pltpu
from jax.experimental.pallas import tpu_sc as plsc

B = 16384
F = 26
EMBD = 32
N = B * F

NC = 2
NS = 16
NW = NC * NS
CHUNK = 128
FT = 4
NBB = B // CHUNK
RAW = FT * NBB * 8
A_ROWS = 3 * NBB * 8
A_PER_W = A_ROWS // NW
B_PER_W = (RAW - A_ROWS) // NW
SUP = 2
N_SUP = 52
NBUF = 4
LANES = 16


def _raw_view(x):
    return (
        jnp.pad(x.T, ((0, 6), (0, 0)))
        .reshape(FT, 8, NBB, CHUNK)
        .transpose(0, 2, 1, 3)
        .reshape(RAW, CHUNK)
    )


TW = 32768
TGRID = (1000001 + TW - 1) // TW
VPAD = TGRID * TW


def _permuted_table(w):
    @functools.partial(
        pl.pallas_call,
        grid=(TGRID,),
        in_specs=[pl.BlockSpec((EMBD, TW), lambda g: (0, g))],
        out_specs=pl.BlockSpec((TW // 4, 128), lambda g: (g, 0)),
        out_shape=jax.ShapeDtypeStruct((VPAD // 4, 128), jnp.float32),
        compiler_params=pltpu.CompilerParams(
            dimension_semantics=("parallel",)
        ),
    )
    def tkern(in_ref, out_ref):
        x = in_ref[...]
        q = TW // 4
        out_ref[...] = jnp.concatenate(
            [x[:, s * q:(s + 1) * q].T for s in range(4)], axis=1
        )

    return tkern(w.T).reshape(VPAD, EMBD)


def kernel(xi, xv, ca_emb_weight):
    xi_n = _raw_view(xi.astype(jnp.int32))
    xv_n = _raw_view(xv)
    table_rm = _permuted_table(ca_emb_weight)

    mesh = plsc.VectorSubcoreMesh(core_axis_name="c", subcore_axis_name="s")

    @functools.partial(
        pl.kernel,
        out_type=jax.ShapeDtypeStruct((F, EMBD // 8, NBB, 8, CHUNK),
                                      jnp.float32),
        mesh=mesh,
        scratch_types=[
            pltpu.VMEM((A_PER_W + B_PER_W, CHUNK), jnp.int32),
            pltpu.VMEM((A_PER_W + B_PER_W, CHUNK), jnp.float32),
        ]
        + [pltpu.VMEM((SUP, CHUNK, EMBD), jnp.float32) for _ in range(NBUF)]
        + [pltpu.VMEM((SUP, CHUNK, EMBD + 1), jnp.float32)]
        + [pltpu.VMEM((EMBD // 8, SUP, 8, CHUNK), jnp.float32)
           for _ in range(NBUF)]
        + [
            pltpu.SemaphoreType.DMA((NBUF,)),
            pltpu.SemaphoreType.DMA((NBUF,)),
        ],
        compiler_params=pltpu.CompilerParams(
            use_tc_tiling_on_sc=False, needs_layout_passes=False
        ),
    )
    def k(table_hbm, idx_hbm, xv_hbm, out_hbm, idx_v, xv_v, *rest):
        gbuf = rest[:NBUF]
        gpad = rest[NBUF]
        obuf = rest[NBUF + 1:NBUF + 1 + NBUF]
        gsem, wsem = rest[-2], rest[-1]
        wid = lax.axis_index("s") * NC + lax.axis_index("c")

        pltpu.sync_copy(idx_hbm.at[pl.ds(wid * A_PER_W, A_PER_W)],
                        idx_v.at[pl.ds(0, A_PER_W)])
        pltpu.sync_copy(idx_hbm.at[pl.ds(A_ROWS + wid * B_PER_W, B_PER_W)],
                        idx_v.at[pl.ds(A_PER_W, B_PER_W)])
        pltpu.sync_copy(xv_hbm.at[pl.ds(wid * A_PER_W, A_PER_W)],
                        xv_v.at[pl.ds(0, A_PER_W)])
        pltpu.sync_copy(xv_hbm.at[pl.ds(A_ROWS + wid * B_PER_W, B_PER_W)],
                        xv_v.at[pl.ds(A_PER_W, B_PER_W)])

        @plsc.parallel_loop(0, A_PER_W + B_PER_W)
        def _(r):
            for q in range(CHUNK // LANES):
                sl = (r, pl.ds(q * LANES, LANES))
                v = idx_v[sl]
                lg = TW.bit_length() - 1
                h = (
                    lax.shift_left(lax.shift_right_logical(v, lg), lg)
                    | lax.shift_left(lax.bitwise_and(v, TW // 4 - 1), 2)
                    | lax.bitwise_and(lax.shift_right_logical(v, lg - 2), 3)
                )
                idx_v.at[sl][...] = h

        iota16 = lax.iota(jnp.int32, LANES)
        jsplat = [jnp.full((LANES,), j, jnp.int32) for j in range(SUP)]
        esplat = [jnp.full((LANES,), e, jnp.int32) for e in range(EMBD)]

        def sup_info(i):
            is_b = i >= 48
            t = i - 48
            fs = jnp.where(is_b, lax.bitwise_and(t, 1), lax.bitwise_and(i, 7))
            grp16 = jnp.where(is_b, lax.shift_right_logical(t, 1),
                              lax.shift_right_logical(i, 3))
            lb = jnp.where(is_b, A_PER_W, 0) + grp16 * 16 + fs
            rr = jnp.where(is_b, A_ROWS + wid * B_PER_W,
                           wid * A_PER_W) + grp16 * 16 + fs
            f = lax.shift_right_logical(rr, 10) * 8 + fs
            bb0 = lax.shift_right_logical(lax.bitwise_and(rr, 1023), 3)
            return lb, f, bb0

        def start_gathers(i, b):
            lb, _, _ = sup_info(i)
            for j in range(SUP):
                pltpu.async_copy(
                    table_hbm.at[idx_v.at[lb + 8 * j]],
                    gbuf[b].at[j],
                    gsem.at[b],
                )

        def wait_gathers(i, b):
            lb, _, _ = sup_info(i)
            for j in range(SUP):
                pltpu.make_async_copy(
                    table_hbm.at[idx_v.at[lb + 8 * j]],
                    gbuf[b].at[j],
                    gsem.at[b],
                ).wait()

        def out_slice(i, tr):
            _, f, bb0 = sup_info(i)
            return out_hbm.at[f, tr, pl.ds(bb0, SUP)]

        def start_writebacks(i, b):
            for tr in range(EMBD // 8):
                pltpu.async_copy(obuf[b].at[tr], out_slice(i, tr), wsem.at[b])

        def wait_writebacks(i, b):
            for tr in range(EMBD // 8):
                pltpu.make_async_copy(
                    obuf[b].at[tr], out_slice(i, tr), wsem.at[b]
                ).wait()

        def compute(i, b):
            src = gbuf[b]

            @plsc.parallel_loop(0, CHUNK)
            def _(r):
                for j in range(SUP):
                    for hh in range(EMBD // LANES):
                        sl = pl.ds(hh * LANES, LANES)
                        gpad.at[j, r, sl][...] = src[j, r, sl]

            g_ref, o_ref = gpad, obuf[b]
            lb, _, _ = sup_info(i)

            @plsc.parallel_loop(0, CHUNK // LANES, unroll=2)
            def _(bl0):
                lane0 = bl0 * LANES
                row_idx = lane0 + iota16
                xvv = [
                    xv_v[lb + 8 * j, pl.ds(lane0, LANES)] for j in range(SUP)
                ]
                for e in range(EMBD):
                    g = [
                        plsc.load_gather(g_ref, [jsplat[j], row_idx, esplat[e]])
                        for j in range(SUP)
                    ]
                    for j in range(SUP):
                        o_ref.at[e // 8, j, e % 8, pl.ds(lane0, LANES)][...] = (
                            g[j] * xvv[j]
                        )

        for i in range(NBUF - 1):
            start_gathers(i, i)

        @pl.loop(0, N_SUP, step=NBUF)
        def _(i0):
            for b in range(NBUF):
                i = i0 + b
                wait_gathers(i, b)
                @pl.when(i + NBUF - 1 < N_SUP)
                def _():
                    start_gathers(i + NBUF - 1, (b + NBUF - 1) % NBUF)
                @pl.when(i >= NBUF)
                def _():
                    wait_writebacks(i - NBUF, b)
                compute(i, b)
                start_writebacks(i, b)

        for b in range(NBUF):
            wait_writebacks(N_SUP - NBUF + b, b)

    out5d = k(table_rm, xi_n, xv_n)
    return jnp.transpose(out5d, (2, 4, 0, 1, 3)).reshape(B, F, EMBD)
